# baseline (speedup 1.0000x reference)
import functools

import jax
import jax.numpy as jnp
from jax import lax
from jax.experimental import pallas as pl
from jax.experimental.pallas import tpu as pltpu
from jax.experimental.pallas import tpu_sc as plsc

B = 16384
NE = 100000
SP = 20
SR = 12
D = 128
H = 256
MSL = 10
VP = 1024
TSLOT = SP + SR
IW = 128
GL = TSLOT

NC = 2
NS = 16
NW = NC * NS
EPW = B // NW
FLUSH = 64


def _proj_body(src_ref, w_ref, o_ref):
    y = jnp.dot(src_ref[0], w_ref[0], preferred_element_type=jnp.float32)
    lo = jax.lax.bitcast_convert_type(y[:, :D].astype(jnp.bfloat16), jnp.int16)
    hi = jax.lax.bitcast_convert_type(y[:, D:].astype(jnp.bfloat16), jnp.int16)
    o_ref[0] = (lo.astype(jnp.int32) & 0xFFFF) | (hi.astype(jnp.int32) << 16)


def _project(src_all, w1r):
    return pl.pallas_call(
        _proj_body,
        grid=(TSLOT,),
        in_specs=[pl.BlockSpec((1, VP, D), lambda s: ((s >= SP).astype(jnp.int32), 0, 0)),
                  pl.BlockSpec((1, D, H), lambda s: (s, 0, 0))],
        out_specs=pl.BlockSpec((1, VP, D), lambda s: (s, 0, 0)),
        out_shape=jax.ShapeDtypeStruct((TSLOT, VP, D), jnp.int32),
    )(src_all, w1r)


def _projd_body(d_ref, w_ref, o_ref):
    o_ref[0] = jnp.dot(d_ref[...], w_ref[0], preferred_element_type=jnp.float32)


def _project_dist(dist_emb, w1r):
    return pl.pallas_call(
        _projd_body,
        grid=(SP,),
        in_specs=[pl.BlockSpec((MSL, D), lambda s: (0, 0)),
                  pl.BlockSpec((1, D, H), lambda s: (s, 0, 0))],
        out_specs=pl.BlockSpec((1, MSL, H), lambda s: (s, 0, 0)),
        out_shape=jax.ShapeDtypeStruct((SP, MSL, H), jnp.float32),
    )(dist_emb, w1r)


_EBLK = 10000


def _idxt_body(h_ref, d_ref, r_ref, o_ref):
    ca = lax.broadcasted_iota(jnp.int32, (_EBLK, SP), 1)
    cr = lax.broadcasted_iota(jnp.int32, (_EBLK, SR), 1)
    o_ref[:, 0:SP] = h_ref[...] + ca * VP
    o_ref[:, SP:TSLOT] = r_ref[...] + (cr + SP) * VP
    o_ref[:, TSLOT:TSLOT + SP] = d_ref[...]
    o_ref[:, TSLOT + SP:] = jnp.zeros((_EBLK, IW - TSLOT - SP), jnp.int32)


def _build_idxt(hashes, distances, rel_ids):
    return pl.pallas_call(
        _idxt_body,
        grid=(NE // _EBLK,),
        in_specs=[pl.BlockSpec((_EBLK, SP), lambda i: (i, 0)),
                  pl.BlockSpec((_EBLK, SP), lambda i: (i, 0)),
                  pl.BlockSpec((_EBLK, SR), lambda i: (i, 0))],
        out_specs=pl.BlockSpec((_EBLK, IW), lambda i: (i, 0)),
        out_shape=jax.ShapeDtypeStruct((NE, IW), jnp.int32),
    )(hashes, distances, rel_ids)


def _gather_body(ent_hbm, idxt_hbm, t_hbm, out_hbm, dc_hbm,
                 ent_v, idx_v, rows_a, rows_b, stage_v, stage_d, sem_i, sem_a, sem_b):
    cid = lax.axis_index("c")
    sid = lax.axis_index("s")
    wid = sid * NC + cid
    base = wid * EPW

    pltpu.sync_copy(ent_hbm.at[pl.ds(base, EPW)], ent_v)
    pltpu.async_copy(idxt_hbm.at[ent_v], idx_v, sem_i).wait()

    pltpu.async_copy(t_hbm.at[idx_v.at[0, pl.ds(0, GL)]], rows_a, sem_a)
    pltpu.async_copy(t_hbm.at[idx_v.at[1, pl.ds(0, GL)]], rows_b, sem_b)

    def load(buf_ref, r):
        out = []
        for c in range(8):
            v = buf_ref[r, pl.ds(c * 16, 16)]
            out.append(plsc.bitcast(v << 16, jnp.float32))
            out.append(plsc.bitcast(v & jnp.int32(-65536), jnp.float32))
        return out

    def consume(buf_ref, e, srow):
        accs = tuple(load(buf_ref, 0))

        def rbody(r, acc):
            row = load(buf_ref, r)
            return tuple(a + x for a, x in zip(acc, row))

        accs = lax.fori_loop(1, GL, rbody, accs)
        for c in range(8):
            stage_v[srow, pl.ds(c * 16, 16)] = accs[2 * c]
            stage_v[srow, pl.ds(D + c * 16, 16)] = accs[2 * c + 1]
        stage_d[srow, pl.ds(0, 16)] = idx_v[e, pl.ds(TSLOT, 16)]
        stage_d[srow, pl.ds(16, 16)] = idx_v[e, pl.ds(TSLOT + 16, 16)]

    def gbody(g, carry):
        e = 2 * g
        pltpu.make_async_copy(t_hbm.at[idx_v.at[0, pl.ds(0, GL)]], rows_a, sem_a).wait()
        consume(rows_a, e, e % FLUSH)

        @pl.when(g < EPW // 2 - 1)
        def _():
            pltpu.async_copy(t_hbm.at[idx_v.at[e + 2, pl.ds(0, GL)]], rows_a, sem_a)

        pltpu.make_async_copy(t_hbm.at[idx_v.at[1, pl.ds(0, GL)]], rows_b, sem_b).wait()
        consume(rows_b, e + 1, (e + 1) % FLUSH)

        @pl.when(g < EPW // 2 - 1)
        def _():
            pltpu.async_copy(t_hbm.at[idx_v.at[e + 3, pl.ds(0, GL)]], rows_b, sem_b)

        @pl.when((e + 1) % FLUSH == FLUSH - 1)
        def _():
            blk = (e + 1) // FLUSH
            pltpu.sync_copy(stage_v, out_hbm.at[pl.ds(base + blk * FLUSH, FLUSH)])
            pltpu.sync_copy(stage_d, dc_hbm.at[pl.ds(base + blk * FLUSH, FLUSH)])

        return carry

    lax.fori_loop(0, EPW // 2, gbody, 0)


@functools.cache
def _gather_kernel():
    return pl.kernel(
        _gather_body,
        mesh=plsc.VectorSubcoreMesh(core_axis_name="c", subcore_axis_name="s"),
        compiler_params=pltpu.CompilerParams(needs_layout_passes=False),
        out_type=(jax.ShapeDtypeStruct((B, H), jnp.float32),
                  jax.ShapeDtypeStruct((B, 32), jnp.int32)),
        scratch_types=[
            pltpu.VMEM((EPW,), jnp.int32),
            pltpu.VMEM((EPW, IW), jnp.int32),
            pltpu.VMEM((GL, D), jnp.int32),
            pltpu.VMEM((GL, D), jnp.int32),
            pltpu.VMEM((FLUSH, H), jnp.float32),
            pltpu.VMEM((FLUSH, 32), jnp.int32),
            pltpu.SemaphoreType.DMA,
            pltpu.SemaphoreType.DMA,
            pltpu.SemaphoreType.DMA,
        ],
    )


_MBLK = 1024


def _mlp_body(x_ref, dc_ref, dt_ref, b1_ref, w2_ref, b2_ref, o_ref):
    d = dc_ref[...][:, :SP]
    d3 = lax.broadcast_in_dim(d, (_MBLK, SP, MSL), (0, 1))
    k3 = lax.broadcasted_iota(jnp.int32, (_MBLK, SP, MSL), 2)
    oh = (d3 == k3).astype(jnp.float32).reshape(_MBLK, SP * MSL)
    x = x_ref[...] + jnp.dot(oh, dt_ref[...], preferred_element_type=jnp.float32)
    h = jnp.maximum(x + b1_ref[...], 0.0)
    o_ref[...] = jnp.dot(h, w2_ref[...], preferred_element_type=jnp.float32) + b2_ref[...]


def _mlp(hidden, dcode, dt, b1, w2, b2):
    return pl.pallas_call(
        _mlp_body,
        grid=(B // _MBLK,),
        in_specs=[pl.BlockSpec((_MBLK, H), lambda i: (i, 0)),
                  pl.BlockSpec((_MBLK, 32), lambda i: (i, 0)),
                  pl.BlockSpec((SP * MSL, H), lambda i: (0, 0)),
                  pl.BlockSpec((1, H), lambda i: (0, 0)),
                  pl.BlockSpec((H, D), lambda i: (0, 0)),
                  pl.BlockSpec((1, D), lambda i: (0, 0))],
        out_specs=pl.BlockSpec((_MBLK, D), lambda i: (i, 0)),
        out_shape=jax.ShapeDtypeStruct((B, D), jnp.float32),
    )(hidden, dcode, dt, b1, w2, b2)


def kernel(entities, hashes, distances, rel_ids, anchor_emb, dist_emb, rel_emb,
           W1, b1, W2, b2):
    ents = entities.astype(jnp.int32)
    pad = lambda a: jnp.pad(a, ((0, VP - a.shape[0]), (0, 0)))
    src_all = jnp.stack([pad(anchor_emb), pad(rel_emb)])
    w1r = W1.reshape(SP + SR, D, H)
    t = _project(src_all, w1r).reshape(TSLOT * VP, D)
    dt = _project_dist(dist_emb, w1r).reshape(SP * MSL, H)
    idxt = _build_idxt(hashes, distances, rel_ids)
    hidden, dcode = _gather_kernel()(ents, idxt, t)
    return _mlp(hidden, dcode, dt, b1.reshape(1, H), W2, b2.reshape(1, D))


# speedup vs baseline: 1.2618x; 1.2618x over previous
import functools

import jax
import jax.numpy as jnp
from jax import lax
from jax.experimental import pallas as pl
from jax.experimental.pallas import tpu as pltpu
from jax.experimental.pallas import tpu_sc as plsc

B = 16384
NE = 100000
SP = 20
SR = 12
D = 128
H = 256
MSL = 10
VP = 1024
TSLOT = SP + SR
IW = 128
GL = TSLOT

NC = 2
NS = 16
NW = NC * NS
EPW = B // NW
FLUSH = 64


def _proj_body(src_ref, w_ref, o_ref):
    y = jnp.dot(src_ref[0], w_ref[0], preferred_element_type=jnp.float32)
    lo = jax.lax.bitcast_convert_type(y[:, :D].astype(jnp.bfloat16), jnp.int16)
    hi = jax.lax.bitcast_convert_type(y[:, D:].astype(jnp.bfloat16), jnp.int16)
    o_ref[0] = (lo.astype(jnp.int32) & 0xFFFF) | (hi.astype(jnp.int32) << 16)


def _project(src_all, w1r):
    return pl.pallas_call(
        _proj_body,
        grid=(TSLOT,),
        in_specs=[pl.BlockSpec((1, VP, D), lambda s: ((s >= SP).astype(jnp.int32), 0, 0)),
                  pl.BlockSpec((1, D, H), lambda s: (s, 0, 0))],
        out_specs=pl.BlockSpec((1, VP, D), lambda s: (s, 0, 0)),
        out_shape=jax.ShapeDtypeStruct((TSLOT, VP, D), jnp.int32),
    )(src_all, w1r)


def _projd_body(d_ref, w_ref, o_ref):
    o_ref[0] = jnp.dot(d_ref[...], w_ref[0], preferred_element_type=jnp.float32)


def _project_dist(dist_emb, w1r):
    return pl.pallas_call(
        _projd_body,
        grid=(SP,),
        in_specs=[pl.BlockSpec((MSL, D), lambda s: (0, 0)),
                  pl.BlockSpec((1, D, H), lambda s: (s, 0, 0))],
        out_specs=pl.BlockSpec((1, MSL, H), lambda s: (s, 0, 0)),
        out_shape=jax.ShapeDtypeStruct((SP, MSL, H), jnp.float32),
    )(dist_emb, w1r)


_EBLK = 10000


def _idxt_body(h_ref, d_ref, r_ref, o_ref):
    ca = lax.broadcasted_iota(jnp.int32, (_EBLK, SP), 1)
    cr = lax.broadcasted_iota(jnp.int32, (_EBLK, SR), 1)
    o_ref[:, 0:SP] = h_ref[...] + ca * VP
    o_ref[:, SP:TSLOT] = r_ref[...] + (cr + SP) * VP
    o_ref[:, TSLOT:TSLOT + SP] = d_ref[...]
    o_ref[:, TSLOT + SP:] = jnp.zeros((_EBLK, IW - TSLOT - SP), jnp.int32)


def _build_idxt(hashes, distances, rel_ids):
    return pl.pallas_call(
        _idxt_body,
        grid=(NE // _EBLK,),
        in_specs=[pl.BlockSpec((_EBLK, SP), lambda i: (i, 0)),
                  pl.BlockSpec((_EBLK, SP), lambda i: (i, 0)),
                  pl.BlockSpec((_EBLK, SR), lambda i: (i, 0))],
        out_specs=pl.BlockSpec((_EBLK, IW), lambda i: (i, 0)),
        out_shape=jax.ShapeDtypeStruct((NE, IW), jnp.int32),
    )(hashes, distances, rel_ids)


NBUF = 4


def _gather_body(ent_hbm, idxt_hbm, t_hbm, out_hbm, dc_hbm,
                 ent_v, idx_v, rows, stage_v, stage_d, sem_i, *sems):
    cid = lax.axis_index("c")
    sid = lax.axis_index("s")
    wid = sid * NC + cid
    base = wid * EPW

    pltpu.sync_copy(ent_hbm.at[pl.ds(base, EPW)], ent_v)
    pltpu.async_copy(idxt_hbm.at[ent_v], idx_v, sem_i).wait()

    for b in range(NBUF):
        pltpu.async_copy(t_hbm.at[idx_v.at[b, pl.ds(0, GL)]], rows.at[b], sems[b])

    def load(buf_ref, r):
        out = []
        for c in range(8):
            v = buf_ref[r, pl.ds(c * 16, 16)]
            out.append(plsc.bitcast(v << 16, jnp.float32))
            out.append(plsc.bitcast(v & jnp.int32(-65536), jnp.float32))
        return out

    def consume(buf_ref, e, srow):
        accs = tuple(load(buf_ref, 0))

        def rbody(r, acc):
            row = load(buf_ref, r)
            return tuple(a + x for a, x in zip(acc, row))

        accs = lax.fori_loop(1, GL, rbody, accs)
        for c in range(8):
            stage_v[srow, pl.ds(c * 16, 16)] = accs[2 * c]
            stage_v[srow, pl.ds(D + c * 16, 16)] = accs[2 * c + 1]
        stage_d[srow, pl.ds(0, 16)] = idx_v[e, pl.ds(TSLOT, 16)]
        stage_d[srow, pl.ds(16, 16)] = idx_v[e, pl.ds(TSLOT + 16, 16)]

    def gbody(g, carry):
        e0 = NBUF * g
        for b in range(NBUF):
            e = e0 + b
            pltpu.make_async_copy(
                t_hbm.at[idx_v.at[b, pl.ds(0, GL)]], rows.at[b], sems[b]).wait()
            consume(rows.at[b], e, e % FLUSH)

            @pl.when(g < EPW // NBUF - 1)
            def _():
                pltpu.async_copy(
                    t_hbm.at[idx_v.at[e + NBUF, pl.ds(0, GL)]], rows.at[b], sems[b])

        @pl.when((e0 + NBUF - 1) % FLUSH == FLUSH - 1)
        def _():
            blk = (e0 + NBUF - 1) // FLUSH
            pltpu.sync_copy(stage_v, out_hbm.at[pl.ds(base + blk * FLUSH, FLUSH)])
            pltpu.sync_copy(stage_d, dc_hbm.at[pl.ds(base + blk * FLUSH, FLUSH)])

        return carry

    lax.fori_loop(0, EPW // NBUF, gbody, 0)


@functools.cache
def _gather_kernel():
    return pl.kernel(
        _gather_body,
        mesh=plsc.VectorSubcoreMesh(core_axis_name="c", subcore_axis_name="s"),
        compiler_params=pltpu.CompilerParams(needs_layout_passes=False),
        out_type=(jax.ShapeDtypeStruct((B, H), jnp.float32),
                  jax.ShapeDtypeStruct((B, 32), jnp.int32)),
        scratch_types=[
            pltpu.VMEM((EPW,), jnp.int32),
            pltpu.VMEM((EPW, IW), jnp.int32),
            pltpu.VMEM((NBUF, GL, D), jnp.int32),
            pltpu.VMEM((FLUSH, H), jnp.float32),
            pltpu.VMEM((FLUSH, 32), jnp.int32),
        ] + [pltpu.SemaphoreType.DMA] * (1 + NBUF),
    )


_MBLK = 1024


def _mlp_body(x_ref, dc_ref, dt_ref, b1_ref, w2_ref, b2_ref, o_ref):
    d = dc_ref[...][:, :SP]
    d3 = lax.broadcast_in_dim(d, (_MBLK, SP, MSL), (0, 1))
    k3 = lax.broadcasted_iota(jnp.int32, (_MBLK, SP, MSL), 2)
    oh = (d3 == k3).astype(jnp.float32).reshape(_MBLK, SP * MSL)
    x = x_ref[...] + jnp.dot(oh, dt_ref[...], preferred_element_type=jnp.float32)
    h = jnp.maximum(x + b1_ref[...], 0.0)
    o_ref[...] = jnp.dot(h, w2_ref[...], preferred_element_type=jnp.float32) + b2_ref[...]


def _mlp(hidden, dcode, dt, b1, w2, b2):
    return pl.pallas_call(
        _mlp_body,
        grid=(B // _MBLK,),
        in_specs=[pl.BlockSpec((_MBLK, H), lambda i: (i, 0)),
                  pl.BlockSpec((_MBLK, 32), lambda i: (i, 0)),
                  pl.BlockSpec((SP * MSL, H), lambda i: (0, 0)),
                  pl.BlockSpec((1, H), lambda i: (0, 0)),
                  pl.BlockSpec((H, D), lambda i: (0, 0)),
                  pl.BlockSpec((1, D), lambda i: (0, 0))],
        out_specs=pl.BlockSpec((_MBLK, D), lambda i: (i, 0)),
        out_shape=jax.ShapeDtypeStruct((B, D), jnp.float32),
    )(hidden, dcode, dt, b1, w2, b2)


def kernel(entities, hashes, distances, rel_ids, anchor_emb, dist_emb, rel_emb,
           W1, b1, W2, b2):
    ents = entities.astype(jnp.int32)
    pad = lambda a: jnp.pad(a, ((0, VP - a.shape[0]), (0, 0)))
    src_all = jnp.stack([pad(anchor_emb), pad(rel_emb)])
    w1r = W1.reshape(SP + SR, D, H)
    t = _project(src_all, w1r).reshape(TSLOT * VP, D)
    dt = _project_dist(dist_emb, w1r).reshape(SP * MSL, H)
    idxt = _build_idxt(hashes, distances, rel_ids)
    hidden, dcode = _gather_kernel()(ents, idxt, t)
    return _mlp(hidden, dcode, dt, b1.reshape(1, H), W2, b2.reshape(1, D))


# speedup vs baseline: 1.3754x; 1.0900x over previous
import functools

import jax
import jax.numpy as jnp
from jax import lax
from jax.experimental import pallas as pl
from jax.experimental.pallas import tpu as pltpu
from jax.experimental.pallas import tpu_sc as plsc

B = 16384
NE = 100000
SP = 20
SR = 12
D = 128
H = 256
MSL = 10
VP = 1024
TSLOT = SP + SR
IW = 128
GL = TSLOT

NC = 2
NS = 16
NW = NC * NS
EPW = B // NW
FLUSH = 64


def _proj_body(src_ref, w_ref, o_ref):
    y = jnp.dot(src_ref[0], w_ref[0], preferred_element_type=jnp.float32)
    lo = jax.lax.bitcast_convert_type(y[:, :D].astype(jnp.bfloat16), jnp.int16)
    hi = jax.lax.bitcast_convert_type(y[:, D:].astype(jnp.bfloat16), jnp.int16)
    o_ref[0] = (lo.astype(jnp.int32) & 0xFFFF) | (hi.astype(jnp.int32) << 16)


def _project(src_all, w1r):
    return pl.pallas_call(
        _proj_body,
        grid=(TSLOT,),
        in_specs=[pl.BlockSpec((1, VP, D), lambda s: ((s >= SP).astype(jnp.int32), 0, 0)),
                  pl.BlockSpec((1, D, H), lambda s: (s, 0, 0))],
        out_specs=pl.BlockSpec((1, VP, D), lambda s: (s, 0, 0)),
        out_shape=jax.ShapeDtypeStruct((TSLOT, VP, D), jnp.int32),
    )(src_all, w1r)


def _projd_body(d_ref, w_ref, o_ref):
    o_ref[0] = jnp.dot(d_ref[...], w_ref[0], preferred_element_type=jnp.float32)


def _project_dist(dist_emb, w1r):
    return pl.pallas_call(
        _projd_body,
        grid=(SP,),
        in_specs=[pl.BlockSpec((MSL, D), lambda s: (0, 0)),
                  pl.BlockSpec((1, D, H), lambda s: (s, 0, 0))],
        out_specs=pl.BlockSpec((1, MSL, H), lambda s: (s, 0, 0)),
        out_shape=jax.ShapeDtypeStruct((SP, MSL, H), jnp.float32),
    )(dist_emb, w1r)


_EBLK = 10000


def _idxt_body(h_ref, d_ref, r_ref, o_ref):
    ca = lax.broadcasted_iota(jnp.int32, (_EBLK, SP), 1)
    cr = lax.broadcasted_iota(jnp.int32, (_EBLK, SR), 1)
    o_ref[:, 0:SP] = h_ref[...] + ca * VP
    o_ref[:, SP:TSLOT] = r_ref[...] + (cr + SP) * VP
    o_ref[:, TSLOT:TSLOT + SP] = d_ref[...]
    o_ref[:, TSLOT + SP:] = jnp.zeros((_EBLK, IW - TSLOT - SP), jnp.int32)


def _build_idxt(hashes, distances, rel_ids):
    return pl.pallas_call(
        _idxt_body,
        grid=(NE // _EBLK,),
        in_specs=[pl.BlockSpec((_EBLK, SP), lambda i: (i, 0)),
                  pl.BlockSpec((_EBLK, SP), lambda i: (i, 0)),
                  pl.BlockSpec((_EBLK, SR), lambda i: (i, 0))],
        out_specs=pl.BlockSpec((_EBLK, IW), lambda i: (i, 0)),
        out_shape=jax.ShapeDtypeStruct((NE, IW), jnp.int32),
    )(hashes, distances, rel_ids)


NBUF = 8


def _gather_body(ent_hbm, idxt_hbm, t_hbm, out_hbm, dc_hbm,
                 ent_v, idx_v, rows, stage_v, stage_d, sem_i, *sems):
    cid = lax.axis_index("c")
    sid = lax.axis_index("s")
    wid = sid * NC + cid
    base = wid * EPW

    pltpu.sync_copy(ent_hbm.at[pl.ds(base, EPW)], ent_v)
    pltpu.async_copy(idxt_hbm.at[ent_v], idx_v, sem_i).wait()

    for b in range(NBUF):
        pltpu.async_copy(t_hbm.at[idx_v.at[b, pl.ds(0, GL)]], rows.at[b], sems[b])

    def load(buf_ref, r):
        out = []
        for c in range(8):
            v = buf_ref[r, pl.ds(c * 16, 16)]
            out.append(plsc.bitcast(v << 16, jnp.float32))
            out.append(plsc.bitcast(v & jnp.int32(-65536), jnp.float32))
        return out

    def consume(buf_ref, e, srow):
        accs = tuple(load(buf_ref, 0))

        def rbody(r, acc):
            row = load(buf_ref, r)
            return tuple(a + x for a, x in zip(acc, row))

        accs = lax.fori_loop(1, GL, rbody, accs)
        for c in range(8):
            stage_v[srow, pl.ds(c * 16, 16)] = accs[2 * c]
            stage_v[srow, pl.ds(D + c * 16, 16)] = accs[2 * c + 1]
        stage_d[srow, pl.ds(0, 16)] = idx_v[e, pl.ds(TSLOT, 16)]
        stage_d[srow, pl.ds(16, 16)] = idx_v[e, pl.ds(TSLOT + 16, 16)]

    def gbody(g, carry):
        e0 = NBUF * g
        for b in range(NBUF):
            e = e0 + b
            pltpu.make_async_copy(
                t_hbm.at[idx_v.at[b, pl.ds(0, GL)]], rows.at[b], sems[b]).wait()
            consume(rows.at[b], e, e % FLUSH)

            @pl.when(g < EPW // NBUF - 1)
            def _():
                pltpu.async_copy(
                    t_hbm.at[idx_v.at[e + NBUF, pl.ds(0, GL)]], rows.at[b], sems[b])

        @pl.when((e0 + NBUF - 1) % FLUSH == FLUSH - 1)
        def _():
            blk = (e0 + NBUF - 1) // FLUSH
            pltpu.sync_copy(stage_v, out_hbm.at[pl.ds(base + blk * FLUSH, FLUSH)])
            pltpu.sync_copy(stage_d, dc_hbm.at[pl.ds(base + blk * FLUSH, FLUSH)])

        return carry

    lax.fori_loop(0, EPW // NBUF, gbody, 0)


@functools.cache
def _gather_kernel():
    return pl.kernel(
        _gather_body,
        mesh=plsc.VectorSubcoreMesh(core_axis_name="c", subcore_axis_name="s"),
        compiler_params=pltpu.CompilerParams(needs_layout_passes=False),
        out_type=(jax.ShapeDtypeStruct((B, H), jnp.float32),
                  jax.ShapeDtypeStruct((B, 32), jnp.int32)),
        scratch_types=[
            pltpu.VMEM((EPW,), jnp.int32),
            pltpu.VMEM((EPW, IW), jnp.int32),
            pltpu.VMEM((NBUF, GL, D), jnp.int32),
            pltpu.VMEM((FLUSH, H), jnp.float32),
            pltpu.VMEM((FLUSH, 32), jnp.int32),
        ] + [pltpu.SemaphoreType.DMA] * (1 + NBUF),
    )


_MBLK = 1024


def _mlp_body(x_ref, dc_ref, dt_ref, b1_ref, w2_ref, b2_ref, o_ref):
    d = dc_ref[...][:, :SP]
    d3 = lax.broadcast_in_dim(d, (_MBLK, SP, MSL), (0, 1))
    k3 = lax.broadcasted_iota(jnp.int32, (_MBLK, SP, MSL), 2)
    oh = (d3 == k3).astype(jnp.float32).reshape(_MBLK, SP * MSL)
    x = x_ref[...] + jnp.dot(oh, dt_ref[...], preferred_element_type=jnp.float32)
    h = jnp.maximum(x + b1_ref[...], 0.0)
    o_ref[...] = jnp.dot(h, w2_ref[...], preferred_element_type=jnp.float32) + b2_ref[...]


def _mlp(hidden, dcode, dt, b1, w2, b2):
    return pl.pallas_call(
        _mlp_body,
        grid=(B // _MBLK,),
        in_specs=[pl.BlockSpec((_MBLK, H), lambda i: (i, 0)),
                  pl.BlockSpec((_MBLK, 32), lambda i: (i, 0)),
                  pl.BlockSpec((SP * MSL, H), lambda i: (0, 0)),
                  pl.BlockSpec((1, H), lambda i: (0, 0)),
                  pl.BlockSpec((H, D), lambda i: (0, 0)),
                  pl.BlockSpec((1, D), lambda i: (0, 0))],
        out_specs=pl.BlockSpec((_MBLK, D), lambda i: (i, 0)),
        out_shape=jax.ShapeDtypeStruct((B, D), jnp.float32),
    )(hidden, dcode, dt, b1, w2, b2)


def kernel(entities, hashes, distances, rel_ids, anchor_emb, dist_emb, rel_emb,
           W1, b1, W2, b2):
    ents = entities.astype(jnp.int32)
    pad = lambda a: jnp.pad(a, ((0, VP - a.shape[0]), (0, 0)))
    src_all = jnp.stack([pad(anchor_emb), pad(rel_emb)])
    w1r = W1.reshape(SP + SR, D, H)
    t = _project(src_all, w1r).reshape(TSLOT * VP, D)
    dt = _project_dist(dist_emb, w1r).reshape(SP * MSL, H)
    idxt = _build_idxt(hashes, distances, rel_ids)
    hidden, dcode = _gather_kernel()(ents, idxt, t)
    return _mlp(hidden, dcode, dt, b1.reshape(1, H), W2, b2.reshape(1, D))


# speedup vs baseline: 1.6575x; 1.2051x over previous
import functools

import jax
import jax.numpy as jnp
import numpy as np
from jax import lax
from jax.experimental import pallas as pl
from jax.experimental.pallas import tpu as pltpu
from jax.experimental.pallas import tpu_sc as plsc

B = 16384
NE = 100000
SP = 20
SR = 12
D = 128
H = 256
MSL = 10
VP = 1024
TSLOT = SP + SR
IW = 128
GL = TSLOT

NC = 2
NS = 16
NW = NC * NS
EPW = B // NW
FLUSH = 64


def _proj_body(src_ref, w_ref, o_ref):
    y = jnp.dot(src_ref[0], w_ref[0], preferred_element_type=jnp.float32)
    lo = jax.lax.bitcast_convert_type(y[:, :D].astype(jnp.bfloat16), jnp.int16)
    hi = jax.lax.bitcast_convert_type(y[:, D:].astype(jnp.bfloat16), jnp.int16)
    o_ref[0] = (lo.astype(jnp.int32) & 0xFFFF) | (hi.astype(jnp.int32) << 16)


def _project(src_all, w1r):
    return pl.pallas_call(
        _proj_body,
        grid=(TSLOT,),
        in_specs=[pl.BlockSpec((1, VP, D), lambda s: ((s >= SP).astype(jnp.int32), 0, 0)),
                  pl.BlockSpec((1, D, H), lambda s: (s, 0, 0))],
        out_specs=pl.BlockSpec((1, VP, D), lambda s: (s, 0, 0)),
        out_shape=jax.ShapeDtypeStruct((TSLOT, VP, D), jnp.int32),
    )(src_all, w1r)


def _projd_body(d_ref, w_ref, o_ref):
    o_ref[0] = jnp.dot(d_ref[...], w_ref[0], preferred_element_type=jnp.float32)


def _project_dist(dist_emb, w1r):
    return pl.pallas_call(
        _projd_body,
        grid=(SP,),
        in_specs=[pl.BlockSpec((MSL, D), lambda s: (0, 0)),
                  pl.BlockSpec((1, D, H), lambda s: (s, 0, 0))],
        out_specs=pl.BlockSpec((1, MSL, H), lambda s: (s, 0, 0)),
        out_shape=jax.ShapeDtypeStruct((SP, MSL, H), jnp.float32),
    )(dist_emb, w1r)


_EBLK = 10000


def _idxt_body(h_ref, d_ref, r_ref, o_ref):
    ca = lax.broadcasted_iota(jnp.int32, (_EBLK, SP), 1)
    cr = lax.broadcasted_iota(jnp.int32, (_EBLK, SR), 1)
    o_ref[:, 0:SP] = h_ref[...] + ca * VP
    o_ref[:, SP:TSLOT] = r_ref[...] + (cr + SP) * VP
    o_ref[:, TSLOT:TSLOT + SP] = d_ref[...]
    o_ref[:, TSLOT + SP:] = jnp.zeros((_EBLK, IW - TSLOT - SP), jnp.int32)


def _build_idxt(hashes, distances, rel_ids):
    return pl.pallas_call(
        _idxt_body,
        grid=(NE // _EBLK,),
        in_specs=[pl.BlockSpec((_EBLK, SP), lambda i: (i, 0)),
                  pl.BlockSpec((_EBLK, SP), lambda i: (i, 0)),
                  pl.BlockSpec((_EBLK, SR), lambda i: (i, 0))],
        out_specs=pl.BlockSpec((_EBLK, IW), lambda i: (i, 0)),
        out_shape=jax.ShapeDtypeStruct((NE, IW), jnp.int32),
    )(hashes, distances, rel_ids)


NBUF = 8


def _gather_body(ent_hbm, idxt_hbm, t_hbm, out_hbm, dc_hbm,
                 ent_v, idx_v, rows, stage_v, stage_d, sem_i, *sems):
    cid = lax.axis_index("c")
    sid = lax.axis_index("s")
    wid = sid * NC + cid
    base = wid * EPW

    pltpu.sync_copy(ent_hbm.at[pl.ds(base, EPW)], ent_v)
    pltpu.async_copy(idxt_hbm.at[ent_v], idx_v, sem_i).wait()

    for b in range(NBUF):
        pltpu.async_copy(t_hbm.at[idx_v.at[b, pl.ds(0, GL)]], rows.at[b], sems[b])

    def load(buf_ref, r):
        out = []
        for c in range(8):
            v = buf_ref[r, pl.ds(c * 16, 16)]
            out.append(plsc.bitcast(v << 16, jnp.float32))
            out.append(plsc.bitcast(v & jnp.int32(-65536), jnp.float32))
        return out

    def consume(buf_ref, e, srow):
        accs = tuple(load(buf_ref, 0))

        def rbody(r, acc):
            row = load(buf_ref, r)
            return tuple(a + x for a, x in zip(acc, row))

        accs = lax.fori_loop(1, GL, rbody, accs)
        for c in range(8):
            stage_v[srow, pl.ds(c * 16, 16)] = accs[2 * c]
            stage_v[srow, pl.ds(D + c * 16, 16)] = accs[2 * c + 1]
        stage_d[srow, pl.ds(0, 16)] = idx_v[e, pl.ds(TSLOT, 16)].astype(jnp.float32)
        stage_d[srow, pl.ds(16, 16)] = idx_v[e, pl.ds(TSLOT + 16, 16)].astype(jnp.float32)

    def gbody(g, carry):
        e0 = NBUF * g
        for b in range(NBUF):
            e = e0 + b
            pltpu.make_async_copy(
                t_hbm.at[idx_v.at[b, pl.ds(0, GL)]], rows.at[b], sems[b]).wait()
            consume(rows.at[b], e, e % FLUSH)

            @pl.when(g < EPW // NBUF - 1)
            def _():
                pltpu.async_copy(
                    t_hbm.at[idx_v.at[e + NBUF, pl.ds(0, GL)]], rows.at[b], sems[b])

        @pl.when((e0 + NBUF - 1) % FLUSH == FLUSH - 1)
        def _():
            blk = (e0 + NBUF - 1) // FLUSH
            pltpu.sync_copy(stage_v, out_hbm.at[pl.ds(base + blk * FLUSH, FLUSH)])
            pltpu.sync_copy(stage_d, dc_hbm.at[pl.ds(base + blk * FLUSH, FLUSH)])

        return carry

    lax.fori_loop(0, EPW // NBUF, gbody, 0)


@functools.cache
def _gather_kernel():
    return pl.kernel(
        _gather_body,
        mesh=plsc.VectorSubcoreMesh(core_axis_name="c", subcore_axis_name="s"),
        compiler_params=pltpu.CompilerParams(needs_layout_passes=False),
        out_type=(jax.ShapeDtypeStruct((B, H), jnp.float32),
                  jax.ShapeDtypeStruct((B, 32), jnp.float32)),
        scratch_types=[
            pltpu.VMEM((EPW,), jnp.int32),
            pltpu.VMEM((EPW, IW), jnp.int32),
            pltpu.VMEM((NBUF, GL, D), jnp.int32),
            pltpu.VMEM((FLUSH, H), jnp.float32),
            pltpu.VMEM((FLUSH, 32), jnp.float32),
        ] + [pltpu.SemaphoreType.DMA] * (1 + NBUF),
    )


_MBLK = 1024


def _mlp_body(x_ref, dc_ref, rrep_ref, kvec_ref, dt_ref, b1_ref, w2_ref, b2_ref, o_ref):
    drep = jnp.dot(dc_ref[...], rrep_ref[...], preferred_element_type=jnp.float32)
    oh = (drep == kvec_ref[...]).astype(jnp.float32)
    x = x_ref[...] + jnp.dot(oh, dt_ref[...], preferred_element_type=jnp.float32)
    h = jnp.maximum(x + b1_ref[...], 0.0)
    o_ref[...] = jnp.dot(h, w2_ref[...], preferred_element_type=jnp.float32) + b2_ref[...]


def _mlp(hidden, dcode, rrep, kvec, dt, b1, w2, b2):
    return pl.pallas_call(
        _mlp_body,
        grid=(B // _MBLK,),
        in_specs=[pl.BlockSpec((_MBLK, H), lambda i: (i, 0)),
                  pl.BlockSpec((_MBLK, 32), lambda i: (i, 0)),
                  pl.BlockSpec((32, SP * MSL), lambda i: (0, 0)),
                  pl.BlockSpec((1, SP * MSL), lambda i: (0, 0)),
                  pl.BlockSpec((SP * MSL, H), lambda i: (0, 0)),
                  pl.BlockSpec((1, H), lambda i: (0, 0)),
                  pl.BlockSpec((H, D), lambda i: (0, 0)),
                  pl.BlockSpec((1, D), lambda i: (0, 0))],
        out_specs=pl.BlockSpec((_MBLK, D), lambda i: (i, 0)),
        out_shape=jax.ShapeDtypeStruct((B, D), jnp.float32),
    )(hidden, dcode, rrep, kvec, dt, b1, w2, b2)


def kernel(entities, hashes, distances, rel_ids, anchor_emb, dist_emb, rel_emb,
           W1, b1, W2, b2):
    ents = entities.astype(jnp.int32)
    pad = lambda a: jnp.pad(a, ((0, VP - a.shape[0]), (0, 0)))
    src_all = jnp.stack([pad(anchor_emb), pad(rel_emb)])
    w1r = W1.reshape(SP + SR, D, H)
    t = _project(src_all, w1r).reshape(TSLOT * VP, D)
    dt = _project_dist(dist_emb, w1r).reshape(SP * MSL, H)
    idxt = _build_idxt(hashes, distances, rel_ids)
    hidden, dcode = _gather_kernel()(ents, idxt, t)
    rrep = np.zeros((32, SP * MSL), np.float32)
    rrep[np.arange(SP * MSL) // MSL, np.arange(SP * MSL)] = 1.0
    kvec = (np.arange(SP * MSL) % MSL).astype(np.float32).reshape(1, SP * MSL)
    return _mlp(hidden, dcode, jnp.asarray(rrep), jnp.asarray(kvec), dt,
                b1.reshape(1, H), W2, b2.reshape(1, D))


# speedup vs baseline: 1.6590x; 1.0009x over previous
import functools

import jax
import jax.numpy as jnp
import numpy as np
from jax import lax
from jax.experimental import pallas as pl
from jax.experimental.pallas import tpu as pltpu
from jax.experimental.pallas import tpu_sc as plsc

B = 16384
NE = 100000
SP = 20
SR = 12
D = 128
H = 256
MSL = 10
VP = 1024
TSLOT = SP + SR
IW = 128
GL = TSLOT

NC = 2
NS = 16
NW = NC * NS
EPW = B // NW
FLUSH = 64


def _proj_body(src_ref, w_ref, d_ref, o_ref, od_ref):
    y = jnp.dot(src_ref[0], w_ref[0], preferred_element_type=jnp.float32)
    lo = jax.lax.bitcast_convert_type(y[:, :D].astype(jnp.bfloat16), jnp.int16)
    hi = jax.lax.bitcast_convert_type(y[:, D:].astype(jnp.bfloat16), jnp.int16)
    o_ref[0] = (lo.astype(jnp.int32) & 0xFFFF) | (hi.astype(jnp.int32) << 16)
    od_ref[0] = jnp.dot(d_ref[...], w_ref[0], preferred_element_type=jnp.float32)


def _project(src_all, w1r, dist_emb):
    return pl.pallas_call(
        _proj_body,
        grid=(TSLOT,),
        in_specs=[pl.BlockSpec((1, VP, D), lambda s: ((s >= SP).astype(jnp.int32), 0, 0)),
                  pl.BlockSpec((1, D, H), lambda s: (s, 0, 0)),
                  pl.BlockSpec((MSL, D), lambda s: (0, 0))],
        out_specs=[pl.BlockSpec((1, VP, D), lambda s: (s, 0, 0)),
                   pl.BlockSpec((1, MSL, H), lambda s: (s, 0, 0))],
        out_shape=[jax.ShapeDtypeStruct((TSLOT, VP, D), jnp.int32),
                   jax.ShapeDtypeStruct((TSLOT, MSL, H), jnp.float32)],
    )(src_all, w1r, dist_emb)


_EBLK = 10000


def _idxt_body(h_ref, d_ref, r_ref, o_ref):
    ca = lax.broadcasted_iota(jnp.int32, (_EBLK, SP), 1)
    cr = lax.broadcasted_iota(jnp.int32, (_EBLK, SR), 1)
    o_ref[:, 0:SP] = h_ref[...] + ca * VP
    o_ref[:, SP:TSLOT] = r_ref[...] + (cr + SP) * VP
    o_ref[:, TSLOT:TSLOT + SP] = d_ref[...]
    o_ref[:, TSLOT + SP:] = jnp.zeros((_EBLK, IW - TSLOT - SP), jnp.int32)


def _build_idxt(hashes, distances, rel_ids):
    return pl.pallas_call(
        _idxt_body,
        grid=(NE // _EBLK,),
        in_specs=[pl.BlockSpec((_EBLK, SP), lambda i: (i, 0)),
                  pl.BlockSpec((_EBLK, SP), lambda i: (i, 0)),
                  pl.BlockSpec((_EBLK, SR), lambda i: (i, 0))],
        out_specs=pl.BlockSpec((_EBLK, IW), lambda i: (i, 0)),
        out_shape=jax.ShapeDtypeStruct((NE, IW), jnp.int32),
    )(hashes, distances, rel_ids)


NBUF = 8


def _gather_body(ent_hbm, idxt_hbm, t_hbm, out_hbm, dc_hbm,
                 ent_v, idx_v, rows, stage_v, stage_d, sem_i, *sems):
    cid = lax.axis_index("c")
    sid = lax.axis_index("s")
    wid = sid * NC + cid
    base = wid * EPW

    pltpu.sync_copy(ent_hbm.at[pl.ds(base, EPW)], ent_v)
    pltpu.async_copy(idxt_hbm.at[ent_v], idx_v, sem_i).wait()

    for b in range(NBUF):
        pltpu.async_copy(t_hbm.at[idx_v.at[b, pl.ds(0, GL)]], rows.at[b], sems[b])

    def load(buf_ref, r):
        out = []
        for c in range(8):
            v = buf_ref[r, pl.ds(c * 16, 16)]
            out.append(plsc.bitcast(v << 16, jnp.float32))
            out.append(plsc.bitcast(v & jnp.int32(-65536), jnp.float32))
        return out

    def consume(buf_ref, e, srow):
        accs = tuple(load(buf_ref, 0))

        def rbody(r, acc):
            row = load(buf_ref, r)
            return tuple(a + x for a, x in zip(acc, row))

        accs = lax.fori_loop(1, GL, rbody, accs)
        for c in range(8):
            stage_v[srow, pl.ds(c * 16, 16)] = accs[2 * c]
            stage_v[srow, pl.ds(D + c * 16, 16)] = accs[2 * c + 1]
        stage_d[srow, pl.ds(0, 16)] = idx_v[e, pl.ds(TSLOT, 16)].astype(jnp.float32)
        stage_d[srow, pl.ds(16, 16)] = idx_v[e, pl.ds(TSLOT + 16, 16)].astype(jnp.float32)

    def gbody(g, carry):
        e0 = NBUF * g
        for b in range(NBUF):
            e = e0 + b
            pltpu.make_async_copy(
                t_hbm.at[idx_v.at[b, pl.ds(0, GL)]], rows.at[b], sems[b]).wait()
            consume(rows.at[b], e, e % FLUSH)

            @pl.when(g < EPW // NBUF - 1)
            def _():
                pltpu.async_copy(
                    t_hbm.at[idx_v.at[e + NBUF, pl.ds(0, GL)]], rows.at[b], sems[b])

        @pl.when((e0 + NBUF - 1) % FLUSH == FLUSH - 1)
        def _():
            blk = (e0 + NBUF - 1) // FLUSH
            pltpu.sync_copy(stage_v, out_hbm.at[pl.ds(base + blk * FLUSH, FLUSH)])
            pltpu.sync_copy(stage_d, dc_hbm.at[pl.ds(base + blk * FLUSH, FLUSH)])

        return carry

    lax.fori_loop(0, EPW // NBUF, gbody, 0)


@functools.cache
def _gather_kernel():
    return pl.kernel(
        _gather_body,
        mesh=plsc.VectorSubcoreMesh(core_axis_name="c", subcore_axis_name="s"),
        compiler_params=pltpu.CompilerParams(needs_layout_passes=False),
        out_type=(jax.ShapeDtypeStruct((B, H), jnp.float32),
                  jax.ShapeDtypeStruct((B, 32), jnp.float32)),
        scratch_types=[
            pltpu.VMEM((EPW,), jnp.int32),
            pltpu.VMEM((EPW, IW), jnp.int32),
            pltpu.VMEM((NBUF, GL, D), jnp.int32),
            pltpu.VMEM((FLUSH, H), jnp.float32),
            pltpu.VMEM((FLUSH, 32), jnp.float32),
        ] + [pltpu.SemaphoreType.DMA] * (1 + NBUF),
    )


_MBLK = 1024


def _mlp_body(x_ref, dc_ref, rrep_ref, kvec_ref, dt_ref, b1_ref, w2_ref, b2_ref, o_ref):
    drep = jnp.dot(dc_ref[...], rrep_ref[...], preferred_element_type=jnp.float32)
    oh = (drep == kvec_ref[...]).astype(jnp.float32)
    x = x_ref[...] + jnp.dot(oh, dt_ref[...], preferred_element_type=jnp.float32)
    h = jnp.maximum(x + b1_ref[...], 0.0)
    o_ref[...] = jnp.dot(h, w2_ref[...], preferred_element_type=jnp.float32) + b2_ref[...]


def _mlp(hidden, dcode, rrep, kvec, dt, b1, w2, b2):
    return pl.pallas_call(
        _mlp_body,
        grid=(B // _MBLK,),
        in_specs=[pl.BlockSpec((_MBLK, H), lambda i: (i, 0)),
                  pl.BlockSpec((_MBLK, 32), lambda i: (i, 0)),
                  pl.BlockSpec((32, SP * MSL), lambda i: (0, 0)),
                  pl.BlockSpec((1, SP * MSL), lambda i: (0, 0)),
                  pl.BlockSpec((SP * MSL, H), lambda i: (0, 0)),
                  pl.BlockSpec((1, H), lambda i: (0, 0)),
                  pl.BlockSpec((H, D), lambda i: (0, 0)),
                  pl.BlockSpec((1, D), lambda i: (0, 0))],
        out_specs=pl.BlockSpec((_MBLK, D), lambda i: (i, 0)),
        out_shape=jax.ShapeDtypeStruct((B, D), jnp.float32),
    )(hidden, dcode, rrep, kvec, dt, b1, w2, b2)


def kernel(entities, hashes, distances, rel_ids, anchor_emb, dist_emb, rel_emb,
           W1, b1, W2, b2):
    ents = entities.astype(jnp.int32)
    pad = lambda a: jnp.pad(a, ((0, VP - a.shape[0]), (0, 0)))
    src_all = jnp.stack([pad(anchor_emb), pad(rel_emb)])
    w1r = W1.reshape(SP + SR, D, H)
    t, dt_full = _project(src_all, w1r, dist_emb)
    t = t.reshape(TSLOT * VP, D)
    dt = dt_full[:SP].reshape(SP * MSL, H)
    idxt = _build_idxt(hashes, distances, rel_ids)
    hidden, dcode = _gather_kernel()(ents, idxt, t)
    rrep = np.zeros((32, SP * MSL), np.float32)
    rrep[np.arange(SP * MSL) // MSL, np.arange(SP * MSL)] = 1.0
    kvec = (np.arange(SP * MSL) % MSL).astype(np.float32).reshape(1, SP * MSL)
    return _mlp(hidden, dcode, jnp.asarray(rrep), jnp.asarray(kvec), dt,
                b1.reshape(1, H), W2, b2.reshape(1, D))
